# baseline (speedup 1.0000x reference)
import functools

import jax
import jax.numpy as jnp
from jax import lax
from jax.experimental import pallas as pl
from jax.experimental.pallas import tpu as pltpu
from jax.experimental.pallas import tpu_sc as plsc


def _make_gather(B, V, D):
    info = plsc.get_sparse_core_info()
    NC, NS = info.num_cores, info.num_subcores
    NW = NC * NS
    assert B % NW == 0
    b_per_w = B // NW
    CHUNK = 64
    NBUF = 4
    assert b_per_w % (CHUNK * NBUF) == 0
    n_chunks = b_per_w // CHUNK
    n_groups = n_chunks // NBUF

    mesh = plsc.VectorSubcoreMesh(core_axis_name="c", subcore_axis_name="s")

    @functools.partial(
        pl.kernel,
        mesh=mesh,
        out_type=jax.ShapeDtypeStruct((B, D), jnp.float32),
        scratch_types=[
            pltpu.VMEM((NBUF * CHUNK,), jnp.int32),
            pltpu.VMEM((NBUF, CHUNK, D), jnp.float32),
            pltpu.VMEM_SHARED((V, D), jnp.float32),
            pltpu.SemaphoreType.DMA((NBUF,)),
            pltpu.SemaphoreType.DMA((NBUF,)),
            pltpu.SemaphoreType.DMA((NBUF,)),
        ],
    )
    def gather_kernel(
        items_hbm, table_hbm, out_hbm, idx_v, rows_v, table_sh, sem_g, sem_o, sem_i
    ):
        wid = lax.axis_index("s") * NC + lax.axis_index("c")
        base = wid * b_per_w
        sid = lax.axis_index("s")
        v_per_s = V // NS
        pltpu.sync_copy(
            table_hbm.at[pl.ds(sid * v_per_s, v_per_s)],
            table_sh.at[pl.ds(sid * v_per_s, v_per_s)],
        )
        plsc.subcore_barrier()

        def start_idx(chunk, b):
            idx = idx_v.at[pl.ds(b * CHUNK, CHUNK)]
            pltpu.async_copy(
                items_hbm.at[pl.ds(base + chunk * CHUNK, CHUNK)], idx, sem_i.at[b]
            )

        def wait_idx(chunk, b):
            idx = idx_v.at[pl.ds(b * CHUNK, CHUNK)]
            pltpu.make_async_copy(
                items_hbm.at[pl.ds(base + chunk * CHUNK, CHUNK)], idx, sem_i.at[b]
            ).wait()

        def start_gather(chunk, b):
            idx = idx_v.at[pl.ds(b * CHUNK, CHUNK)]
            return pltpu.async_copy(table_sh.at[idx], rows_v.at[b], sem_g.at[b])

        def wait_gather(chunk, b):
            idx = idx_v.at[pl.ds(b * CHUNK, CHUNK)]
            pltpu.make_async_copy(table_sh.at[idx], rows_v.at[b], sem_g.at[b]).wait()

        def start_out(chunk, b):
            dst = out_hbm.at[pl.ds(base + chunk * CHUNK, CHUNK)]
            return pltpu.async_copy(rows_v.at[b], dst, sem_o.at[b])

        def wait_out(chunk, b):
            dst = out_hbm.at[pl.ds(base + chunk * CHUNK, CHUNK)]
            pltpu.make_async_copy(rows_v.at[b], dst, sem_o.at[b]).wait()

        for b in range(NBUF):
            start_idx(b, b)
        for b in range(NBUF):
            wait_idx(b, b)
            start_gather(b, b)

        def group_body(g, carry):
            for b in range(NBUF):
                i = g * NBUF + b
                wait_gather(i, b)
                start_idx(i + NBUF, b)
                start_out(i, b)
                wait_out(i, b)
                wait_idx(i + NBUF, b)
                start_gather(i + NBUF, b)
            return carry

        lax.fori_loop(0, n_groups - 1, group_body, 0)

        for b in range(NBUF):
            i = (n_groups - 1) * NBUF + b
            wait_gather(i, b)
            start_out(i, b)
            wait_out(i, b)

    return gather_kernel


def kernel(items, embed):
    B = items.shape[0]
    V, D = embed.shape
    items = items.astype(jnp.int32)
    embed = embed.astype(jnp.float32)
    return _make_gather(B, V, D)(items, embed)


# speedup vs baseline: 1.1727x; 1.1727x over previous
import functools

import jax
import jax.numpy as jnp
from jax import lax
from jax.experimental import pallas as pl
from jax.experimental.pallas import tpu as pltpu
from jax.experimental.pallas import tpu_sc as plsc


def _make_gather(B, V, D):
    info = plsc.get_sparse_core_info()
    NC, NS = info.num_cores, info.num_subcores
    NW = NC * NS
    assert B % NW == 0
    b_per_w = B // NW
    CHUNK = 160
    NBUF = 2
    assert b_per_w % (CHUNK * NBUF) == 0
    n_chunks = b_per_w // CHUNK
    n_groups = n_chunks // NBUF

    mesh = plsc.VectorSubcoreMesh(core_axis_name="c", subcore_axis_name="s")

    @functools.partial(
        pl.kernel,
        mesh=mesh,
        out_type=jax.ShapeDtypeStruct((B, D), jnp.float32),
        scratch_types=[
            pltpu.VMEM((NBUF * CHUNK,), jnp.int32),
            pltpu.VMEM((NBUF, CHUNK, D), jnp.float32),
            pltpu.VMEM_SHARED((V, D), jnp.float32),
            pltpu.SemaphoreType.DMA((NBUF,)),
            pltpu.SemaphoreType.DMA((NBUF,)),
            pltpu.SemaphoreType.DMA((NBUF,)),
        ],
    )
    def gather_kernel(
        items_hbm, table_hbm, out_hbm, idx_v, rows_v, table_sh, sem_g, sem_o, sem_i
    ):
        wid = lax.axis_index("s") * NC + lax.axis_index("c")
        base = wid * b_per_w
        sid = lax.axis_index("s")
        v_per_s = V // NS
        pltpu.sync_copy(
            table_hbm.at[pl.ds(sid * v_per_s, v_per_s)],
            table_sh.at[pl.ds(sid * v_per_s, v_per_s)],
        )
        plsc.subcore_barrier()

        def start_idx(chunk, b):
            idx = idx_v.at[pl.ds(b * CHUNK, CHUNK)]
            pltpu.async_copy(
                items_hbm.at[pl.ds(base + chunk * CHUNK, CHUNK)], idx, sem_i.at[b]
            )

        def wait_idx(chunk, b):
            idx = idx_v.at[pl.ds(b * CHUNK, CHUNK)]
            pltpu.make_async_copy(
                items_hbm.at[pl.ds(base + chunk * CHUNK, CHUNK)], idx, sem_i.at[b]
            ).wait()

        def start_gather(chunk, b):
            idx = idx_v.at[pl.ds(b * CHUNK, CHUNK)]
            return pltpu.async_copy(table_sh.at[idx], rows_v.at[b], sem_g.at[b])

        def wait_gather(chunk, b):
            idx = idx_v.at[pl.ds(b * CHUNK, CHUNK)]
            pltpu.make_async_copy(table_sh.at[idx], rows_v.at[b], sem_g.at[b]).wait()

        def start_out(chunk, b):
            dst = out_hbm.at[pl.ds(base + chunk * CHUNK, CHUNK)]
            return pltpu.async_copy(rows_v.at[b], dst, sem_o.at[b])

        def wait_out(chunk, b):
            dst = out_hbm.at[pl.ds(base + chunk * CHUNK, CHUNK)]
            pltpu.make_async_copy(rows_v.at[b], dst, sem_o.at[b]).wait()

        for b in range(NBUF):
            start_idx(b, b)
        for b in range(NBUF):
            wait_idx(b, b)
            start_gather(b, b)

        def group_body(g, carry):
            for b in range(NBUF):
                i = g * NBUF + b
                wait_gather(i, b)
                start_idx(i + NBUF, b)
                start_out(i, b)
                wait_out(i, b)
                wait_idx(i + NBUF, b)
                start_gather(i + NBUF, b)
            return carry

        lax.fori_loop(0, n_groups - 1, group_body, 0)

        for b in range(NBUF):
            i = (n_groups - 1) * NBUF + b
            wait_gather(i, b)
            start_out(i, b)
            wait_out(i, b)

    return gather_kernel


def kernel(items, embed):
    B = items.shape[0]
    V, D = embed.shape
    items = items.astype(jnp.int32)
    embed = embed.astype(jnp.float32)
    return _make_gather(B, V, D)(items, embed)


# speedup vs baseline: 1.1802x; 1.0064x over previous
import functools

import jax
import jax.numpy as jnp
from jax import lax
from jax.experimental import pallas as pl
from jax.experimental.pallas import tpu as pltpu
from jax.experimental.pallas import tpu_sc as plsc


def _make_gather(B, V, D):
    info = plsc.get_sparse_core_info()
    NC, NS = info.num_cores, info.num_subcores
    NW = NC * NS
    assert B % NW == 0
    b_per_w = B // NW
    CHUNK = 200
    NBUF = 2
    assert b_per_w % (CHUNK * NBUF) == 0
    n_chunks = b_per_w // CHUNK
    n_groups = n_chunks // NBUF

    mesh = plsc.VectorSubcoreMesh(core_axis_name="c", subcore_axis_name="s")

    @functools.partial(
        pl.kernel,
        mesh=mesh,
        out_type=jax.ShapeDtypeStruct((B, D), jnp.float32),
        scratch_types=[
            pltpu.VMEM((NBUF * CHUNK,), jnp.int32),
            pltpu.VMEM((NBUF, CHUNK, D), jnp.float32),
            pltpu.VMEM_SHARED((V, D), jnp.float32),
            pltpu.SemaphoreType.DMA((NBUF,)),
            pltpu.SemaphoreType.DMA((NBUF,)),
            pltpu.SemaphoreType.DMA((NBUF,)),
        ],
    )
    def gather_kernel(
        items_hbm, table_hbm, out_hbm, idx_v, rows_v, table_sh, sem_g, sem_o, sem_i
    ):
        wid = lax.axis_index("s") * NC + lax.axis_index("c")
        base = wid * b_per_w
        def start_idx(chunk, b):
            idx = idx_v.at[pl.ds(b * CHUNK, CHUNK)]
            pltpu.async_copy(
                items_hbm.at[pl.ds(base + chunk * CHUNK, CHUNK)], idx, sem_i.at[b]
            )

        def wait_idx(chunk, b):
            idx = idx_v.at[pl.ds(b * CHUNK, CHUNK)]
            pltpu.make_async_copy(
                items_hbm.at[pl.ds(base + chunk * CHUNK, CHUNK)], idx, sem_i.at[b]
            ).wait()

        def start_gather(chunk, b):
            idx = idx_v.at[pl.ds(b * CHUNK, CHUNK)]
            return pltpu.async_copy(table_sh.at[idx], rows_v.at[b], sem_g.at[b])

        def wait_gather(chunk, b):
            idx = idx_v.at[pl.ds(b * CHUNK, CHUNK)]
            pltpu.make_async_copy(table_sh.at[idx], rows_v.at[b], sem_g.at[b]).wait()

        def start_out(chunk, b):
            dst = out_hbm.at[pl.ds(base + chunk * CHUNK, CHUNK)]
            return pltpu.async_copy(rows_v.at[b], dst, sem_o.at[b])

        def wait_out(chunk, b):
            dst = out_hbm.at[pl.ds(base + chunk * CHUNK, CHUNK)]
            pltpu.make_async_copy(rows_v.at[b], dst, sem_o.at[b]).wait()

        for b in range(NBUF):
            start_idx(b, b)
        sid = lax.axis_index("s")
        v_per_s = V // NS
        pltpu.sync_copy(
            table_hbm.at[pl.ds(sid * v_per_s, v_per_s)],
            table_sh.at[pl.ds(sid * v_per_s, v_per_s)],
        )
        plsc.subcore_barrier()
        for b in range(NBUF):
            wait_idx(b, b)
            start_gather(b, b)

        def group_body(g, carry):
            for b in range(NBUF):
                i = g * NBUF + b
                wait_gather(i, b)
                start_idx(i + NBUF, b)
                start_out(i, b)
                wait_out(i, b)
                wait_idx(i + NBUF, b)
                start_gather(i + NBUF, b)
            return carry

        lax.fori_loop(0, n_groups - 1, group_body, 0)

        for b in range(NBUF):
            i = (n_groups - 1) * NBUF + b
            wait_gather(i, b)
            start_out(i, b)
            wait_out(i, b)

    return gather_kernel


def kernel(items, embed):
    B = items.shape[0]
    V, D = embed.shape
    items = items.astype(jnp.int32)
    embed = embed.astype(jnp.float32)
    return _make_gather(B, V, D)(items, embed)
